# baseline (speedup 1.0000x reference)
import functools

import jax
import jax.numpy as jnp
from jax import lax
from jax.experimental import pallas as pl
from jax.experimental.pallas import tpu as pltpu
from jax.experimental.pallas import tpu_sc as plsc

f32 = jnp.float32

FC = 128
NCHUNK = 4
EB = 128
TILES = 16
BN = 400


def _sc_degree(dst_pad, ones_eb, zeros_deg, np_rows):
    epad = dst_pad.shape[0]
    nb = epad // (2 * TILES * EB)
    mesh = plsc.VectorSubcoreMesh(core_axis_name="c", subcore_axis_name="s")

    def body(dst_hbm, ones_hbm, zeros_hbm, deg0, deg1, dst_v, ones_v, zeros_v,
             acc):
        c = lax.axis_index("c")
        s = lax.axis_index("s")
        pltpu.sync_copy(ones_hbm, ones_v)
        pltpu.sync_copy(zeros_hbm, zeros_v)

        @pl.when(s == 0)
        def _():
            def zstep(i, carry):
                pltpu.sync_copy(zeros_v, acc.at[pl.ds(i * EB, EB)])
                return carry
            lax.fori_loop(0, np_rows // EB, zstep, 0)

        plsc.subcore_barrier()
        ebase = (c * TILES + s) * (nb * EB)

        def step(i, carry):
            b = pl.multiple_of(ebase + i * EB, EB)
            pltpu.sync_copy(dst_hbm.at[pl.ds(b, EB)], dst_v)
            pltpu.sync_copy(ones_v, acc.at[dst_v], add=True)
            return carry

        lax.fori_loop(0, nb, step, 0)
        plsc.subcore_barrier()

        @pl.when(jnp.logical_and(c == 0, s == 0))
        def _():
            def wstep(i, carry):
                sl = pl.ds(i * EB, EB)
                pltpu.sync_copy(acc.at[sl], zeros_v)
                pltpu.sync_copy(zeros_v, deg0.at[sl])
                return carry
            lax.fori_loop(0, np_rows // EB, wstep, 0)

        @pl.when(jnp.logical_and(c == 1, s == 0))
        def _():
            def wstep(i, carry):
                sl = pl.ds(i * EB, EB)
                pltpu.sync_copy(acc.at[sl], zeros_v)
                pltpu.sync_copy(zeros_v, deg1.at[sl])
                return carry
            lax.fori_loop(0, np_rows // EB, wstep, 0)

    ker = pl.kernel(
        body,
        out_type=(jax.ShapeDtypeStruct((np_rows, FC), f32),
                  jax.ShapeDtypeStruct((np_rows, FC), f32)),
        mesh=mesh,
        scratch_types=[
            pltpu.VMEM((EB,), jnp.int32),
            pltpu.VMEM((EB, FC), f32),
            pltpu.VMEM((EB, FC), f32),
            pltpu.VMEM_SHARED((np_rows, FC), f32),
        ],
    )
    return ker(dst_pad, ones_eb, zeros_deg)


def _sc_aggregate(src_pad, dst_pad, zeros_blk, t_chunks, np_rows):
    epad = src_pad.shape[0]
    nb = epad // (TILES * EB)
    mesh = plsc.VectorSubcoreMesh(core_axis_name="c", subcore_axis_name="s")

    nh = nb // 2

    def body(src_hbm, dst_hbm, zeros_hbm, t0h, t1h, t2h, t3h,
             o0, o1, o2, o3, src_v, dst_v, rb0, rb1, acc,
             zsem, gs0, gs1, ss0, ss1):
        c = lax.axis_index("c")
        s = lax.axis_index("s")
        rbs = (rb0, rb1)
        gss = (gs0, gs1)
        sss = (ss0, ss1)

        def do_chunk(t_ref, o_ref):
            @pl.when(s == 0)
            def _():
                pltpu.sync_copy(zeros_hbm, rb0)

                def zwin(i, carry):
                    for k in range(8):
                        pltpu.async_copy(
                            rb0, acc.at[pl.ds((i * 8 + k) * EB, EB)], zsem)
                    for k in range(8):
                        pltpu.make_async_copy(
                            rb0, acc.at[pl.ds((i * 8 + k) * EB, EB)],
                            zsem).wait()
                    return carry
                lax.fori_loop(0, np_rows // (8 * EB), zwin, 0)

            plsc.subcore_barrier()

            def gather(j, b):
                pltpu.async_copy(t_ref.at[src_v.at[j]], rbs[b], gss[b])

            def scatter(j, b):
                pltpu.async_copy(rbs[b], acc.at[dst_v.at[j]], sss[b], add=True)

            def wait_gather(b):
                pltpu.make_async_copy(t_ref.at[src_v.at[0]], rbs[b],
                                      gss[b]).wait()

            def wait_scatter(b):
                pltpu.make_async_copy(rbs[b], acc.at[dst_v.at[0]],
                                      sss[b]).wait()

            for h in range(2):
                pltpu.sync_copy(src_hbm.at[pl.ds(s * nb + h * nh, nh)], src_v)
                pltpu.sync_copy(dst_hbm.at[pl.ds(s * nb + h * nh, nh)], dst_v)
                gather(0, 0)
                gather(1, 1)

                def step(i, carry):
                    wait_gather(0)
                    scatter(2 * i, 0)
                    wait_gather(1)
                    scatter(2 * i + 1, 1)
                    wait_scatter(0)
                    gather(2 * i + 2, 0)
                    wait_scatter(1)
                    gather(2 * i + 3, 1)
                    return carry

                lax.fori_loop(0, nh // 2 - 1, step, 0)
                wait_gather(0)
                scatter(nh - 2, 0)
                wait_gather(1)
                scatter(nh - 1, 1)
                wait_scatter(0)
                wait_scatter(1)
            plsc.subcore_barrier()

            nblk = np_rows // EB

            @pl.when(s == 0)
            def _():
                pltpu.async_copy(acc.at[pl.ds(0, EB)], rb0, gs0)

                def wpair(i, carry):
                    b0 = pl.ds((2 * i) * EB, EB)
                    b1 = pl.ds((2 * i + 1) * EB, EB)
                    pltpu.make_async_copy(acc.at[b0], rb0, gs0).wait()
                    pltpu.async_copy(rb0, o_ref.at[b0], ss0)
                    pltpu.async_copy(acc.at[b1], rb1, gs1)
                    pltpu.make_async_copy(acc.at[b1], rb1, gs1).wait()
                    pltpu.make_async_copy(rb0, o_ref.at[b0], ss0).wait()
                    pltpu.async_copy(rb1, o_ref.at[b1], ss1)
                    nxt = lax.rem(2 * i + 2, nblk)
                    pltpu.async_copy(acc.at[pl.ds(nxt * EB, EB)], rb0, gs0)
                    pltpu.make_async_copy(rb1, o_ref.at[b1], ss1).wait()
                    return carry

                lax.fori_loop(0, nblk // 2, wpair, 0)
                pltpu.make_async_copy(acc.at[pl.ds(0, EB)], rb0, gs0).wait()

            plsc.subcore_barrier()

        @pl.when(c == 0)
        def _():
            do_chunk(t0h, o0)
            do_chunk(t1h, o1)

        @pl.when(c == 1)
        def _():
            do_chunk(t2h, o2)
            do_chunk(t3h, o3)

    ker = pl.kernel(
        body,
        out_type=tuple(jax.ShapeDtypeStruct((np_rows, FC), f32)
                       for _ in range(NCHUNK)),
        mesh=mesh,
        scratch_types=[
            pltpu.VMEM((nh, EB), jnp.int32),
            pltpu.VMEM((nh, EB), jnp.int32),
            pltpu.VMEM((EB, FC), f32),
            pltpu.VMEM((EB, FC), f32),
            pltpu.VMEM_SHARED((np_rows, FC), f32),
        ] + [pltpu.SemaphoreType.DMA] * 5,
    )
    return ker(src_pad.reshape(TILES * nb, EB), dst_pad.reshape(TILES * nb, EB),
               zeros_blk, *t_chunks)


def _mm(a, w):
    return lax.dot_general(a, w, (((1,), (1,)), ((), ())),
                           preferred_element_type=f32)


def _tc_h0(x, W_enc, b_enc):
    n, d_in = x.shape
    h = W_enc.shape[0]

    def body(x_ref, we_ref, be_ref, o_ref):
        o_ref[...] = _mm(x_ref[...], we_ref[...]) + be_ref[...]

    return pl.pallas_call(
        body,
        grid=(n // BN,),
        in_specs=[
            pl.BlockSpec((BN, d_in), lambda i: (i, 0)),
            pl.BlockSpec((h, d_in), lambda i: (0, 0)),
            pl.BlockSpec((1, h), lambda i: (0, 0)),
        ],
        out_specs=pl.BlockSpec((BN, h), lambda i: (i, 0)),
        out_shape=jax.ShapeDtypeStruct((n, h), f32),
    )(x, W_enc, b_enc.reshape(1, h))


def _tc_t1(h0, deg0, deg1, W_g1, np_rows):
    n, h = h0.shape
    nblk = n // BN

    def body(h0_ref, d0_ref, d1_ref, w1_ref, t0, t1, t2, t3, dv_ref):
        deg = d0_ref[:, 0:1] + d1_ref[:, 0:1] + 1.0
        dinv = lax.rsqrt(deg)
        t = dinv * _mm(h0_ref[...], w1_ref[...])
        t0[...] = t[:, 0 * FC:1 * FC]
        t1[...] = t[:, 1 * FC:2 * FC]
        t2[...] = t[:, 2 * FC:3 * FC]
        t3[...] = t[:, 3 * FC:4 * FC]
        dv_ref[...] = dinv

    outs = pl.pallas_call(
        body,
        grid=(nblk,),
        in_specs=[
            pl.BlockSpec((BN, h), lambda i: (i, 0)),
            pl.BlockSpec((BN, FC), lambda i: (i, 0)),
            pl.BlockSpec((BN, FC), lambda i: (i, 0)),
            pl.BlockSpec((h, h), lambda i: (0, 0)),
        ],
        out_specs=[pl.BlockSpec((BN, FC), lambda i: (i, 0))
                   for _ in range(NCHUNK)] + [pl.BlockSpec((BN, 1), lambda i: (i, 0))],
        out_shape=[jax.ShapeDtypeStruct((np_rows, FC), f32)
                   for _ in range(NCHUNK)] + [jax.ShapeDtypeStruct((n, 1), f32)],
    )(h0, deg0, deg1, W_g1)
    return outs[:NCHUNK], outs[NCHUNK]


def _tc_mid(accs, ts, dinv, b_prev, W, np_rows):
    n = dinv.shape[0]
    h = W.shape[0]
    nblk = n // BN

    def body(a0, a1, a2, a3, t0, t1, t2, t3, dv_ref, b_ref, w_ref,
             o0, o1, o2, o3):
        dv = dv_ref[...]
        b = b_ref[...]
        ar = (a0, a1, a2, a3)
        tr = (t0, t1, t2, t3)
        xs = [jnp.maximum(dv * (ar[f][...] + tr[f][...]) + b[:, f * FC:(f + 1) * FC], 0.0)
              for f in range(NCHUNK)]
        X = jnp.concatenate(xs, axis=1)
        Y = dv * _mm(X, w_ref[...])
        o0[...] = Y[:, 0 * FC:1 * FC]
        o1[...] = Y[:, 1 * FC:2 * FC]
        o2[...] = Y[:, 2 * FC:3 * FC]
        o3[...] = Y[:, 3 * FC:4 * FC]

    chunk_spec = pl.BlockSpec((BN, FC), lambda i: (i, 0))
    outs = pl.pallas_call(
        body,
        grid=(nblk,),
        in_specs=[chunk_spec] * 8 + [
            pl.BlockSpec((BN, 1), lambda i: (i, 0)),
            pl.BlockSpec((1, h), lambda i: (0, 0)),
            pl.BlockSpec((h, h), lambda i: (0, 0)),
        ],
        out_specs=[chunk_spec] * NCHUNK,
        out_shape=[jax.ShapeDtypeStruct((np_rows, FC), f32)
                   for _ in range(NCHUNK)],
    )(*accs, *ts, dinv, b_prev.reshape(1, h), W)
    return outs


def _tc_pool(accs, ts, dinv, b_g3, batch_r, g_count):
    n = dinv.shape[0]
    h = NCHUNK * FC
    nblk = n // BN

    def body(a0, a1, a2, a3, t0, t1, t2, t3, dv_ref, b_ref, bat_ref,
             sums_ref, cnt_ref):
        i = pl.program_id(0)
        dv = dv_ref[...]
        b = b_ref[...]
        ar = (a0, a1, a2, a3)
        tr = (t0, t1, t2, t3)
        hs = [dv * (ar[f][...] + tr[f][...]) + b[:, f * FC:(f + 1) * FC]
              for f in range(NCHUNK)]
        h3 = jnp.concatenate(hs, axis=1)
        bm = bat_ref[0, 0, :]
        iot = lax.broadcasted_iota(jnp.int32, (g_count, BN), 0)
        ind = (bm[None, :] == iot).astype(f32)
        part = lax.dot_general(ind, h3, (((1,), (0,)), ((), ())),
                               preferred_element_type=f32)
        pcnt = jnp.sum(ind, axis=1, keepdims=True)

        @pl.when(i == 0)
        def _():
            sums_ref[...] = jnp.zeros_like(sums_ref)
            cnt_ref[...] = jnp.zeros_like(cnt_ref)

        sums_ref[...] += part
        cnt_ref[...] += pcnt

    chunk_spec = pl.BlockSpec((BN, FC), lambda i: (i, 0))
    sums, cnt = pl.pallas_call(
        body,
        grid=(nblk,),
        in_specs=[chunk_spec] * 8 + [
            pl.BlockSpec((BN, 1), lambda i: (i, 0)),
            pl.BlockSpec((1, h), lambda i: (0, 0)),
            pl.BlockSpec((1, 1, BN), lambda i: (i, 0, 0)),
        ],
        out_specs=[pl.BlockSpec((g_count, h), lambda i: (0, 0)),
                   pl.BlockSpec((g_count, 1), lambda i: (0, 0))],
        out_shape=[jax.ShapeDtypeStruct((g_count, h), f32),
                   jax.ShapeDtypeStruct((g_count, 1), f32)],
    )(*accs, *ts, dinv, b_g3.reshape(1, h), batch_r)
    return sums, cnt


def _tc_head(sums, cnt, gf, W_ge1, b_ge1, W_ge2, b_ge2,
             W_p1, b_p1, W_p2, b_p2, W_p3, b_p3):
    g_count = sums.shape[0]

    def body(sums_ref, cnt_ref, gf_ref, wge1, bge1, wge2, bge2,
             wp1, bp1, wp2, bp2, wp3, bp3, out_ref):
        repr_ = sums_ref[...] / jnp.maximum(cnt_ref[...], 1.0)
        g = jnp.maximum(_mm(gf_ref[...], wge1[...]) + bge1[...], 0.0)
        g = _mm(g, wge2[...]) + bge2[...]
        fused = jnp.concatenate([repr_, g], axis=1)
        p = jnp.maximum(_mm(fused, wp1[...]) + bp1[...], 0.0)
        p = jnp.maximum(_mm(p, wp2[...]) + bp2[...], 0.0)
        out_ref[...] = jnp.sum(p * wp3[...], axis=1, keepdims=True) + bp3[...]

    return pl.pallas_call(
        body,
        out_shape=jax.ShapeDtypeStruct((g_count, 1), f32),
    )(sums, cnt, gf, W_ge1, b_ge1.reshape(1, -1), W_ge2, b_ge2.reshape(1, -1),
      W_p1, b_p1.reshape(1, -1), W_p2, b_p2.reshape(1, -1),
      W_p3, b_p3.reshape(1, 1))


def kernel(x, edge_index, global_features, batch,
           W_enc, b_enc, W_g1, b_g1, W_g2, b_g2, W_g3, b_g3,
           W_ge1, b_ge1, W_ge2, b_ge2,
           W_p1, b_p1, W_p2, b_p2, W_p3, b_p3):
    n = x.shape[0]
    e = edge_index.shape[1]
    np_rows = ((n + 2047) // 2048) * 2048
    epad = ((e + 4095) // 4096) * 4096

    src = jnp.concatenate(
        [edge_index[0], jnp.zeros((epad - e,), jnp.int32)])
    dst = jnp.concatenate(
        [edge_index[1], jnp.full((epad - e,), np_rows - 1, jnp.int32)])

    ones_eb = jnp.ones((EB, FC), f32)
    zeros_blk = jnp.zeros((EB, FC), f32)

    h0 = _tc_h0(x, W_enc, b_enc)
    deg0, deg1 = _sc_degree(dst, ones_eb, zeros_blk, np_rows)
    t, dinv = _tc_t1(h0, deg0, deg1, W_g1, np_rows)

    acc = _sc_aggregate(src, dst, zeros_blk, t, np_rows)
    t = _tc_mid(acc, t, dinv, b_g1, W_g2, np_rows)
    acc = _sc_aggregate(src, dst, zeros_blk, t, np_rows)
    t = _tc_mid(acc, t, dinv, b_g2, W_g3, np_rows)
    acc = _sc_aggregate(src, dst, zeros_blk, t, np_rows)

    batch_r = batch.reshape(n // BN, 1, BN)
    sums, cnt = _tc_pool(acc, t, dinv, b_g3, batch_r, global_features.shape[0])
    return _tc_head(sums, cnt, global_features, W_ge1, b_ge1, W_ge2, b_ge2,
                    W_p1, b_p1, W_p2, b_p2, W_p3, b_p3)


# speedup vs baseline: 1.0155x; 1.0155x over previous
import functools

import jax
import jax.numpy as jnp
from jax import lax
from jax.experimental import pallas as pl
from jax.experimental.pallas import tpu as pltpu
from jax.experimental.pallas import tpu_sc as plsc

f32 = jnp.float32

FC = 128
NCHUNK = 4
EB = 128
TILES = 16
BN = 400


def _sc_degree(dst_pad, ones_eb, zeros_deg, np_rows):
    epad = dst_pad.shape[0]
    nb = epad // (2 * TILES * EB)
    mesh = plsc.VectorSubcoreMesh(core_axis_name="c", subcore_axis_name="s")

    def body(dst_hbm, ones_hbm, zeros_hbm, deg0, deg1, dst_v, ones_v, zeros_v,
             acc):
        c = lax.axis_index("c")
        s = lax.axis_index("s")
        pltpu.sync_copy(ones_hbm, ones_v)
        pltpu.sync_copy(zeros_hbm, zeros_v)

        @pl.when(s == 0)
        def _():
            def zstep(i, carry):
                pltpu.sync_copy(zeros_v, acc.at[pl.ds(i * EB, EB)])
                return carry
            lax.fori_loop(0, np_rows // EB, zstep, 0)

        plsc.subcore_barrier()
        ebase = (c * TILES + s) * (nb * EB)

        def step(i, carry):
            b = pl.multiple_of(ebase + i * EB, EB)
            pltpu.sync_copy(dst_hbm.at[pl.ds(b, EB)], dst_v)
            pltpu.sync_copy(ones_v, acc.at[dst_v], add=True)
            return carry

        lax.fori_loop(0, nb, step, 0)
        plsc.subcore_barrier()

        @pl.when(jnp.logical_and(c == 0, s == 0))
        def _():
            def wstep(i, carry):
                sl = pl.ds(i * EB, EB)
                pltpu.sync_copy(acc.at[sl], zeros_v)
                pltpu.sync_copy(zeros_v, deg0.at[sl])
                return carry
            lax.fori_loop(0, np_rows // EB, wstep, 0)

        @pl.when(jnp.logical_and(c == 1, s == 0))
        def _():
            def wstep(i, carry):
                sl = pl.ds(i * EB, EB)
                pltpu.sync_copy(acc.at[sl], zeros_v)
                pltpu.sync_copy(zeros_v, deg1.at[sl])
                return carry
            lax.fori_loop(0, np_rows // EB, wstep, 0)

    ker = pl.kernel(
        body,
        out_type=(jax.ShapeDtypeStruct((np_rows, FC), f32),
                  jax.ShapeDtypeStruct((np_rows, FC), f32)),
        mesh=mesh,
        scratch_types=[
            pltpu.VMEM((EB,), jnp.int32),
            pltpu.VMEM((EB, FC), f32),
            pltpu.VMEM((EB, FC), f32),
            pltpu.VMEM_SHARED((np_rows, FC), f32),
        ],
    )
    return ker(dst_pad, ones_eb, zeros_deg)


def _sc_aggregate(src_pad, dst_pad, zeros_blk, t_chunks, np_rows):
    epad = src_pad.shape[0]
    nb = epad // (TILES * EB)
    mesh = plsc.VectorSubcoreMesh(core_axis_name="c", subcore_axis_name="s")

    nh = nb // 2

    def body(src_hbm, dst_hbm, zeros_hbm, t0h, t1h, t2h, t3h,
             o0, o1, o2, o3, src_v, dst_v, rb0, rb1, acc,
             zsem, gs0, gs1, ss0, ss1):
        c = lax.axis_index("c")
        s = lax.axis_index("s")
        rbs = (rb0, rb1)
        gss = (gs0, gs1)
        sss = (ss0, ss1)

        def do_chunk(t_ref, o_ref):
            @pl.when(s == 0)
            def _():
                pltpu.sync_copy(zeros_hbm, rb0)

                def zwin(i, carry):
                    for k in range(8):
                        pltpu.async_copy(
                            rb0, acc.at[pl.ds((i * 8 + k) * EB, EB)], zsem)
                    for k in range(8):
                        pltpu.make_async_copy(
                            rb0, acc.at[pl.ds((i * 8 + k) * EB, EB)],
                            zsem).wait()
                    return carry
                lax.fori_loop(0, np_rows // (8 * EB), zwin, 0)

            plsc.subcore_barrier()

            def gather(j, b):
                pltpu.async_copy(t_ref.at[src_v.at[j]], rbs[b], gss[b])

            def scatter(j, b):
                pltpu.async_copy(rbs[b], acc.at[dst_v.at[j]], sss[b], add=True)

            def wait_gather(b):
                pltpu.make_async_copy(t_ref.at[src_v.at[0]], rbs[b],
                                      gss[b]).wait()

            def wait_scatter(b):
                pltpu.make_async_copy(rbs[b], acc.at[dst_v.at[0]],
                                      sss[b]).wait()

            for h in range(2):
                pltpu.sync_copy(src_hbm.at[pl.ds(s * nb + h * nh, nh)], src_v)
                pltpu.sync_copy(dst_hbm.at[pl.ds(s * nb + h * nh, nh)], dst_v)
                gather(0, 0)
                gather(1, 1)

                def step(i, carry):
                    wait_gather(0)
                    scatter(2 * i, 0)
                    wait_gather(1)
                    scatter(2 * i + 1, 1)
                    wait_scatter(0)
                    gather(2 * i + 2, 0)
                    wait_scatter(1)
                    gather(2 * i + 3, 1)
                    return carry

                lax.fori_loop(0, nh // 2 - 1, step, 0)
                wait_gather(0)
                scatter(nh - 2, 0)
                wait_gather(1)
                scatter(nh - 1, 1)
                wait_scatter(0)
                wait_scatter(1)
            plsc.subcore_barrier()

            nblk = np_rows // EB

            @pl.when(s == 0)
            def _():
                pltpu.async_copy(acc.at[pl.ds(0, EB)], rb0, gs0)

                def wpair(i, carry):
                    b0 = pl.ds((2 * i) * EB, EB)
                    b1 = pl.ds((2 * i + 1) * EB, EB)
                    pltpu.make_async_copy(acc.at[b0], rb0, gs0).wait()
                    pltpu.async_copy(rb0, o_ref.at[b0], ss0)
                    pltpu.async_copy(acc.at[b1], rb1, gs1)
                    pltpu.make_async_copy(acc.at[b1], rb1, gs1).wait()
                    pltpu.make_async_copy(rb0, o_ref.at[b0], ss0).wait()
                    pltpu.async_copy(rb1, o_ref.at[b1], ss1)
                    nxt = lax.rem(2 * i + 2, nblk)
                    pltpu.async_copy(acc.at[pl.ds(nxt * EB, EB)], rb0, gs0)
                    pltpu.make_async_copy(rb1, o_ref.at[b1], ss1).wait()
                    return carry

                lax.fori_loop(0, nblk // 2, wpair, 0)
                pltpu.make_async_copy(acc.at[pl.ds(0, EB)], rb0, gs0).wait()

            plsc.subcore_barrier()

        @pl.when(c == 0)
        def _():
            do_chunk(t0h, o0)
            do_chunk(t1h, o1)

        @pl.when(c == 1)
        def _():
            do_chunk(t2h, o2)
            do_chunk(t3h, o3)

    ker = pl.kernel(
        body,
        out_type=tuple(jax.ShapeDtypeStruct((np_rows, FC), f32)
                       for _ in range(NCHUNK)),
        mesh=mesh,
        scratch_types=[
            pltpu.VMEM((nh, EB), jnp.int32),
            pltpu.VMEM((nh, EB), jnp.int32),
            pltpu.VMEM((EB, FC), f32),
            pltpu.VMEM((EB, FC), f32),
            pltpu.VMEM_SHARED((np_rows, FC), f32),
        ] + [pltpu.SemaphoreType.DMA] * 5,
    )
    return ker(src_pad.reshape(TILES * nb, EB), dst_pad.reshape(TILES * nb, EB),
               zeros_blk, *t_chunks)


def _mm(a, w):
    return lax.dot_general(a, w, (((1,), (1,)), ((), ())),
                           preferred_element_type=f32)


def _tc_encode(x, deg0, deg1, W_enc, b_enc, W_g1, np_rows):
    n, d_in = x.shape
    h = W_enc.shape[0]
    nblk = n // BN

    def body(x_ref, d0_ref, d1_ref, we_ref, be_ref, w1_ref,
             t0, t1, t2, t3, dv_ref):
        h0 = _mm(x_ref[...], we_ref[...]) + be_ref[...]
        deg = d0_ref[:, 0:1] + d1_ref[:, 0:1] + 1.0
        dinv = lax.rsqrt(deg)
        t = dinv * _mm(h0, w1_ref[...])
        t0[...] = t[:, 0 * FC:1 * FC]
        t1[...] = t[:, 1 * FC:2 * FC]
        t2[...] = t[:, 2 * FC:3 * FC]
        t3[...] = t[:, 3 * FC:4 * FC]
        dv_ref[...] = dinv

    outs = pl.pallas_call(
        body,
        grid=(nblk,),
        in_specs=[
            pl.BlockSpec((BN, d_in), lambda i: (i, 0)),
            pl.BlockSpec((BN, FC), lambda i: (i, 0)),
            pl.BlockSpec((BN, FC), lambda i: (i, 0)),
            pl.BlockSpec((h, d_in), lambda i: (0, 0)),
            pl.BlockSpec((1, h), lambda i: (0, 0)),
            pl.BlockSpec((h, h), lambda i: (0, 0)),
        ],
        out_specs=[pl.BlockSpec((BN, FC), lambda i: (i, 0))
                   for _ in range(NCHUNK)] + [pl.BlockSpec((BN, 1), lambda i: (i, 0))],
        out_shape=[jax.ShapeDtypeStruct((np_rows, FC), f32)
                   for _ in range(NCHUNK)] + [jax.ShapeDtypeStruct((n, 1), f32)],
    )(x, deg0, deg1, W_enc, b_enc.reshape(1, h), W_g1)
    return outs[:NCHUNK], outs[NCHUNK]


def _tc_mid(accs, ts, dinv, b_prev, W, np_rows):
    n = dinv.shape[0]
    h = W.shape[0]
    nblk = n // BN

    def body(a0, a1, a2, a3, t0, t1, t2, t3, dv_ref, b_ref, w_ref,
             o0, o1, o2, o3):
        dv = dv_ref[...]
        b = b_ref[...]
        ar = (a0, a1, a2, a3)
        tr = (t0, t1, t2, t3)
        xs = [jnp.maximum(dv * (ar[f][...] + tr[f][...]) + b[:, f * FC:(f + 1) * FC], 0.0)
              for f in range(NCHUNK)]
        X = jnp.concatenate(xs, axis=1)
        Y = dv * _mm(X, w_ref[...])
        o0[...] = Y[:, 0 * FC:1 * FC]
        o1[...] = Y[:, 1 * FC:2 * FC]
        o2[...] = Y[:, 2 * FC:3 * FC]
        o3[...] = Y[:, 3 * FC:4 * FC]

    chunk_spec = pl.BlockSpec((BN, FC), lambda i: (i, 0))
    outs = pl.pallas_call(
        body,
        grid=(nblk,),
        in_specs=[chunk_spec] * 8 + [
            pl.BlockSpec((BN, 1), lambda i: (i, 0)),
            pl.BlockSpec((1, h), lambda i: (0, 0)),
            pl.BlockSpec((h, h), lambda i: (0, 0)),
        ],
        out_specs=[chunk_spec] * NCHUNK,
        out_shape=[jax.ShapeDtypeStruct((np_rows, FC), f32)
                   for _ in range(NCHUNK)],
    )(*accs, *ts, dinv, b_prev.reshape(1, h), W)
    return outs


def _tc_pool(accs, ts, dinv, b_g3, batch_r, g_count):
    n = dinv.shape[0]
    h = NCHUNK * FC
    nblk = n // BN

    def body(a0, a1, a2, a3, t0, t1, t2, t3, dv_ref, b_ref, bat_ref,
             sums_ref, cnt_ref):
        i = pl.program_id(0)
        dv = dv_ref[...]
        b = b_ref[...]
        ar = (a0, a1, a2, a3)
        tr = (t0, t1, t2, t3)
        hs = [dv * (ar[f][...] + tr[f][...]) + b[:, f * FC:(f + 1) * FC]
              for f in range(NCHUNK)]
        h3 = jnp.concatenate(hs, axis=1)
        bm = bat_ref[0, 0, :]
        iot = lax.broadcasted_iota(jnp.int32, (g_count, BN), 0)
        ind = (bm[None, :] == iot).astype(f32)
        part = lax.dot_general(ind, h3, (((1,), (0,)), ((), ())),
                               preferred_element_type=f32)
        pcnt = jnp.sum(ind, axis=1, keepdims=True)

        @pl.when(i == 0)
        def _():
            sums_ref[...] = jnp.zeros_like(sums_ref)
            cnt_ref[...] = jnp.zeros_like(cnt_ref)

        sums_ref[...] += part
        cnt_ref[...] += pcnt

    chunk_spec = pl.BlockSpec((BN, FC), lambda i: (i, 0))
    sums, cnt = pl.pallas_call(
        body,
        grid=(nblk,),
        in_specs=[chunk_spec] * 8 + [
            pl.BlockSpec((BN, 1), lambda i: (i, 0)),
            pl.BlockSpec((1, h), lambda i: (0, 0)),
            pl.BlockSpec((1, 1, BN), lambda i: (i, 0, 0)),
        ],
        out_specs=[pl.BlockSpec((g_count, h), lambda i: (0, 0)),
                   pl.BlockSpec((g_count, 1), lambda i: (0, 0))],
        out_shape=[jax.ShapeDtypeStruct((g_count, h), f32),
                   jax.ShapeDtypeStruct((g_count, 1), f32)],
    )(*accs, *ts, dinv, b_g3.reshape(1, h), batch_r)
    return sums, cnt


def _tc_head(sums, cnt, gf, W_ge1, b_ge1, W_ge2, b_ge2,
             W_p1, b_p1, W_p2, b_p2, W_p3, b_p3):
    g_count = sums.shape[0]

    def body(sums_ref, cnt_ref, gf_ref, wge1, bge1, wge2, bge2,
             wp1, bp1, wp2, bp2, wp3, bp3, out_ref):
        repr_ = sums_ref[...] / jnp.maximum(cnt_ref[...], 1.0)
        g = jnp.maximum(_mm(gf_ref[...], wge1[...]) + bge1[...], 0.0)
        g = _mm(g, wge2[...]) + bge2[...]
        fused = jnp.concatenate([repr_, g], axis=1)
        p = jnp.maximum(_mm(fused, wp1[...]) + bp1[...], 0.0)
        p = jnp.maximum(_mm(p, wp2[...]) + bp2[...], 0.0)
        out_ref[...] = jnp.sum(p * wp3[...], axis=1, keepdims=True) + bp3[...]

    return pl.pallas_call(
        body,
        out_shape=jax.ShapeDtypeStruct((g_count, 1), f32),
    )(sums, cnt, gf, W_ge1, b_ge1.reshape(1, -1), W_ge2, b_ge2.reshape(1, -1),
      W_p1, b_p1.reshape(1, -1), W_p2, b_p2.reshape(1, -1),
      W_p3, b_p3.reshape(1, 1))


def kernel(x, edge_index, global_features, batch,
           W_enc, b_enc, W_g1, b_g1, W_g2, b_g2, W_g3, b_g3,
           W_ge1, b_ge1, W_ge2, b_ge2,
           W_p1, b_p1, W_p2, b_p2, W_p3, b_p3):
    n = x.shape[0]
    e = edge_index.shape[1]
    np_rows = ((n + 2047) // 2048) * 2048
    epad = ((e + 4095) // 4096) * 4096

    src = jnp.concatenate(
        [edge_index[0], jnp.zeros((epad - e,), jnp.int32)])
    dst = jnp.concatenate(
        [edge_index[1], jnp.full((epad - e,), np_rows - 1, jnp.int32)])

    ones_eb = jnp.ones((EB, FC), f32)
    zeros_blk = jnp.zeros((EB, FC), f32)

    deg0, deg1 = _sc_degree(dst, ones_eb, zeros_blk, np_rows)
    t, dinv = _tc_encode(x, deg0, deg1, W_enc, b_enc, W_g1, np_rows)

    acc = _sc_aggregate(src, dst, zeros_blk, t, np_rows)
    t = _tc_mid(acc, t, dinv, b_g1, W_g2, np_rows)
    acc = _sc_aggregate(src, dst, zeros_blk, t, np_rows)
    t = _tc_mid(acc, t, dinv, b_g2, W_g3, np_rows)
    acc = _sc_aggregate(src, dst, zeros_blk, t, np_rows)

    batch_r = batch.reshape(n // BN, 1, BN)
    sums, cnt = _tc_pool(acc, t, dinv, b_g3, batch_r, global_features.shape[0])
    return _tc_head(sums, cnt, global_features, W_ge1, b_ge1, W_ge2, b_ge2,
                    W_p1, b_p1, W_p2, b_p2, W_p3, b_p3)
